# baseline (speedup 1.0000x reference)
import jax
import jax.numpy as jnp
from jax.experimental import pallas as pl

N = 10000
K = 5
HID = 32


def kernel(timesteps, edge_index, Wx_i, Wh_i, w_c_i, b_i, Wx_f, Wh_f, w_c_f, b_f,
           Wx_c, Wh_c, b_c, Wx_o, Wh_o, w_c_o, b_o, W_lin, b_lin):
    src = edge_index[0]
    dst = edge_index[1]
    mask = (src != dst).astype(jnp.float32)
    deg = jax.ops.segment_sum(mask, src, num_segments=N)
    dis = jnp.where(deg > 0, 1.0 / jnp.sqrt(jnp.maximum(deg, 1e-12)), 0.0)
    w = -dis[src] * mask * dis[dst]

    WxAll = jnp.concatenate([Wx_i, Wx_f, Wx_c, Wx_o], axis=2)
    WhAll = jnp.concatenate([Wh_i, Wh_f, Wh_c, Wh_o], axis=2)

    def lap(x):
        return jax.ops.segment_sum(w[:, None] * jnp.take(x, src, axis=0), dst, num_segments=N)

    def cheb_all(x, Theta):
        Tx0 = x
        out = Tx0 @ Theta[0]
        Tx1 = lap(Tx0)
        out = out + Tx1 @ Theta[1]
        for k in range(2, K):
            Tx2 = 2.0 * lap(Tx1) - Tx0
            out = out + Tx2 @ Theta[k]
            Tx0, Tx1 = Tx1, Tx2
        return out

    H = jnp.zeros((N, HID), dtype=jnp.float32)
    C = jnp.zeros((N, HID), dtype=jnp.float32)
    for t in range(timesteps.shape[0]):
        x = timesteps[t]
        XW = cheb_all(x, WxAll)
        HW = cheb_all(H, WhAll)
        G = XW + HW
        I = jax.nn.sigmoid(G[:, 0:HID] + w_c_i * C + b_i)
        Fg = jax.nn.sigmoid(G[:, HID:2 * HID] + w_c_f * C + b_f)
        Tc = jnp.tanh(G[:, 2 * HID:3 * HID] + b_c)
        C = Fg * C + I * Tc
        O = jax.nn.sigmoid(G[:, 3 * HID:4 * HID] + w_c_o * C + b_o)
        H = O * jnp.tanh(C)
    return jax.nn.relu(H) @ W_lin + b_lin


# speedup vs baseline: 2.4014x; 2.4014x over previous
import functools

import jax
import jax.numpy as jnp
from jax import lax
from jax.experimental import pallas as pl
from jax.experimental.pallas import tpu as pltpu
from jax.experimental.pallas import tpu_sc as plsc

N = 10000
E = 320000
F_IN = 128
HID = 32
K = 5
T = 12

NC = 2
NS = 16
NW = NC * NS
ROWS = 320
NP = NW * ROWS
CH = 128
NCHUNK = E // CH
DR = 64


def _lap_body(first, D, table, srcs, dsts, ws, se, prev, out,
              acc, rows, sidx, didx, wbuf, sev, prevb, outb, sem):
    QD = D // 16
    wid = lax.axis_index("s") * NC + lax.axis_index("c")
    lo = wid * ROWS

    pltpu.sync_copy(se, sev)
    start = sev[pl.ds(wid, 16)][0]
    end = sev[pl.ds(NW + wid, 16)][0]

    def zrow(r, _):
        for q in range(QD):
            acc[r, pl.ds(q * 16, 16)] = jnp.zeros((16,), jnp.float32)
        return 0
    lax.fori_loop(0, ROWS, zrow, 0)

    c0 = start // CH
    c1 = (end + CH - 1) // CH

    def chunk(c, _):
        base = c * CH
        pltpu.sync_copy(srcs.at[pl.ds(base, CH)], sidx)
        pltpu.sync_copy(dsts.at[pl.ds(base, CH)], didx)
        pltpu.sync_copy(ws.at[pl.ds(base, CH)], wbuf)
        pltpu.async_copy(table.at[sidx], rows, sem).wait()

        def edge16(j16, _):
            j0 = j16 * 16
            g16 = base + j0 + lax.iota(jnp.int32, 16)
            valid = jnp.logical_and(g16 >= start, g16 < end)
            w16 = jnp.where(valid, wbuf[pl.ds(j0, 16)], 0.0)
            dl16 = jnp.clip(didx[pl.ds(j0, 16)] - lo, 0, ROWS - 1)
            for l in range(16):
                wv = jnp.full((16,), w16[l], jnp.float32)
                dl = dl16[l]
                for q in range(QD):
                    plsc.addupdate(acc.at[dl, pl.ds(q * 16, 16)],
                                   rows[j0 + l, pl.ds(q * 16, 16)] * wv)
            return 0
        lax.fori_loop(0, CH // 16, edge16, 0)
        return 0
    lax.fori_loop(c0, c1, chunk, 0)

    for p in range(ROWS // DR):
        r0 = lo + p * DR
        if not first:
            pltpu.sync_copy(prev.at[pl.ds(r0, DR)], prevb)

        def drain(r, _):
            for q in range(QD):
                a = acc[p * DR + r, pl.ds(q * 16, 16)]
                if first:
                    res = a
                else:
                    res = 2.0 * a - prevb[r, pl.ds(q * 16, 16)]
                outb[r, pl.ds(q * 16, 16)] = res
            return 0
        lax.fori_loop(0, DR, drain, 0)
        pltpu.sync_copy(outb, out.at[pl.ds(r0, DR)])


@functools.lru_cache(maxsize=None)
def _make_lap(first, D):
    mesh = plsc.VectorSubcoreMesh(core_axis_name="c", subcore_axis_name="s",
                                  num_cores=NC, num_subcores=NS)
    scratch = [
        pltpu.VMEM((ROWS, D), jnp.float32),
        pltpu.VMEM((CH, D), jnp.float32),
        pltpu.VMEM((CH,), jnp.int32),
        pltpu.VMEM((CH,), jnp.int32),
        pltpu.VMEM((CH,), jnp.float32),
        pltpu.VMEM((96,), jnp.int32),
        pltpu.VMEM((DR, D), jnp.float32),
        pltpu.VMEM((DR, D), jnp.float32),
        pltpu.SemaphoreType.DMA,
    ]
    body = functools.partial(_lap_body, first, D)
    params = None
    if D < 128:
        params = pltpu.CompilerParams(use_tc_tiling_on_sc=False)
    return pl.kernel(body,
                     out_type=jax.ShapeDtypeStruct((NP, D), jnp.float32),
                     mesh=mesh, scratch_types=scratch,
                     compiler_params=params,
                     name=f"sc_lap_{'first' if first else 'rec'}_{D}")


def _lap_first(table, srcs, dsts, ws, se, D):
    dummy = table
    return _make_lap(True, D)(table, srcs, dsts, ws, se, dummy)


def _lap_rec(table, prev, srcs, dsts, ws, se, D):
    return _make_lap(False, D)(table, srcs, dsts, ws, se, prev)


def _cheb_stack(x, srcs, dsts, ws, se, D):
    t0 = x
    t1 = _lap_first(t0, srcs, dsts, ws, se, D)
    t2 = _lap_rec(t1, t0, srcs, dsts, ws, se, D)
    t3 = _lap_rec(t2, t1, srcs, dsts, ws, se, D)
    t4 = _lap_rec(t3, t2, srcs, dsts, ws, se, D)
    return [t0, t1, t2, t3, t4]


BLK = 1024
GRID = NP // BLK


def _dense_step_body(t0_flag, tx0, tx1, tx2, tx3, tx4, th0, th1, th2, th3, th4,
                     c_ref, wx, wh, pb, hn, cn):
    g = jnp.dot(tx0[...], wx[0:F_IN, :], preferred_element_type=jnp.float32)
    for k, txk in enumerate((tx1, tx2, tx3, tx4)):
        g += jnp.dot(txk[...], wx[(k + 1) * F_IN:(k + 2) * F_IN, :],
                     preferred_element_type=jnp.float32)
    if not t0_flag:
        for k, thk in enumerate((th0, th1, th2, th3, th4)):
            g += jnp.dot(thk[...], wh[k * HID:(k + 1) * HID, :],
                         preferred_element_type=jnp.float32)
    w_c_i = pb[0:1, :]
    w_c_f = pb[1:2, :]
    w_c_o = pb[2:3, :]
    b_i = pb[3:4, :]
    b_f = pb[4:5, :]
    b_c = pb[5:6, :]
    b_o = pb[6:7, :]
    c = jnp.zeros((BLK, HID), jnp.float32) if t0_flag else c_ref[...]
    gi = g[:, 0:HID]
    gf = g[:, HID:2 * HID]
    gc = g[:, 2 * HID:3 * HID]
    go = g[:, 3 * HID:4 * HID]
    i_t = jax.nn.sigmoid(gi + w_c_i * c + b_i)
    f_t = jax.nn.sigmoid(gf + w_c_f * c + b_f)
    t_c = jnp.tanh(gc + b_c)
    c_new = f_t * c + i_t * t_c
    o_t = jax.nn.sigmoid(go + w_c_o * c_new + b_o)
    cn[...] = c_new
    hn[...] = o_t * jnp.tanh(c_new)


@functools.lru_cache(maxsize=None)
def _make_dense_step(t0_flag):
    big = pl.BlockSpec((BLK, F_IN), lambda i: (i, 0))
    small = pl.BlockSpec((BLK, HID), lambda i: (i, 0))
    wxs = pl.BlockSpec((K * F_IN, 4 * HID), lambda i: (0, 0))
    whs = pl.BlockSpec((K * HID, 4 * HID), lambda i: (0, 0))
    pbs = pl.BlockSpec((8, HID), lambda i: (0, 0))
    return pl.pallas_call(
        functools.partial(_dense_step_body, t0_flag),
        grid=(GRID,),
        in_specs=[big] * 5 + [small] * 5 + [small, wxs, whs, pbs],
        out_specs=[small, small],
        out_shape=[jax.ShapeDtypeStruct((NP, HID), jnp.float32),
                   jax.ShapeDtypeStruct((NP, HID), jnp.float32)],
        name=f"dense_step_{t0_flag}",
    )


def _head_body(h_ref, wl, bl, out):
    out[...] = jnp.dot(jax.nn.relu(h_ref[...]), wl[...],
                       preferred_element_type=jnp.float32) + bl[...]


@functools.lru_cache(maxsize=None)
def _make_head():
    return pl.pallas_call(
        _head_body,
        grid=(GRID,),
        in_specs=[pl.BlockSpec((BLK, HID), lambda i: (i, 0)),
                  pl.BlockSpec((HID, 128), lambda i: (0, 0)),
                  pl.BlockSpec((1, 128), lambda i: (0, 0))],
        out_specs=pl.BlockSpec((BLK, 128), lambda i: (i, 0)),
        out_shape=jax.ShapeDtypeStruct((NP, 128), jnp.float32),
        name="head",
    )


def kernel(timesteps, edge_index, Wx_i, Wh_i, w_c_i, b_i, Wx_f, Wh_f, w_c_f, b_f,
           Wx_c, Wh_c, b_c, Wx_o, Wh_o, w_c_o, b_o, W_lin, b_lin):
    src = edge_index[0]
    dst = edge_index[1]
    mask = src != dst
    deg = jax.ops.segment_sum(mask.astype(jnp.float32), src, num_segments=N)
    dis = jnp.where(deg > 0, 1.0 / jnp.sqrt(jnp.maximum(deg, 1e-12)), 0.0)
    w = jnp.where(mask, -dis[src] * dis[dst], 0.0)

    order = jnp.argsort(dst)
    srcs = src[order].astype(jnp.int32)
    dsts = dst[order].astype(jnp.int32)
    ws = w[order]
    bounds = jnp.arange(NW, dtype=jnp.int32) * ROWS
    starts = jnp.searchsorted(dsts, bounds, side="left").astype(jnp.int32)
    ends = jnp.searchsorted(dsts, bounds + ROWS, side="left").astype(jnp.int32)
    se = jnp.concatenate([starts, ends, jnp.zeros((32,), jnp.int32)])

    WxAll = jnp.concatenate([Wx_i, Wx_f, Wx_c, Wx_o], axis=2).reshape(K * F_IN, 4 * HID)
    WhAll = jnp.concatenate([Wh_i, Wh_f, Wh_c, Wh_o], axis=2).reshape(K * HID, 4 * HID)
    PB = jnp.concatenate([w_c_i, w_c_f, w_c_o, b_i, b_f, b_c, b_o,
                          jnp.zeros((1, HID), jnp.float32)], axis=0)

    xp = jnp.pad(timesteps, ((0, 0), (0, NP - N), (0, 0)))
    W_lin_p = jnp.pad(W_lin, ((0, 0), (0, 128 - W_lin.shape[1])))
    b_lin_p = jnp.pad(b_lin, ((0, 128 - b_lin.shape[0]),)).reshape(1, 128)

    zeros_h = jnp.zeros((NP, HID), jnp.float32)
    H = zeros_h
    C = zeros_h
    for t in range(T):
        txs = _cheb_stack(xp[t], srcs, dsts, ws, se, F_IN)
        if t == 0:
            ths = [zeros_h] * K
        else:
            ths = _cheb_stack(H, srcs, dsts, ws, se, HID)
        H, C = _make_dense_step(t == 0)(*txs, *ths, C, WxAll, WhAll, PB)

    out = _make_head()(H, W_lin_p, b_lin_p)
    return out[:N, :T]


# speedup vs baseline: 3.2093x; 1.3364x over previous
import functools

import jax
import jax.numpy as jnp
from jax import lax
from jax.experimental import pallas as pl
from jax.experimental.pallas import tpu as pltpu
from jax.experimental.pallas import tpu_sc as plsc

N = 10000
E = 320000
F_IN = 128
HID = 32
K = 5
T = 12

NC = 2
NS = 16
NW = NC * NS
ROWS = 320
NP = NW * ROWS
CH = 128
MAXE = 8192
DR = 64


def _lap_body(first, D, table, srcs, dsts, ws, se, prev, out,
              acc, rows, es, ed, ew, sev, prevb, outb, sem0, sem1):
    QD = D // 16
    wid = lax.axis_index("s") * NC + lax.axis_index("c")
    lo = wid * ROWS

    pltpu.sync_copy(se, sev)
    start = sev[pl.ds(wid, 16)][0]
    end = sev[pl.ds(NW + wid, 16)][0]

    start8 = (start // 8) * 8
    n_outer = (end - start8 + MAXE - 1) // MAXE
    sems = (sem0, sem1)

    def zrow(r, _):
        for q in range(QD):
            acc[r, pl.ds(q * 16, 16)] = jnp.zeros((16,), jnp.float32)
        return 0
    lax.fori_loop(0, ROWS, zrow, 0)

    def outer(o, _):
        obase = start8 + o * MAXE
        pltpu.sync_copy(srcs.at[pl.ds(obase, MAXE)], es)
        pltpu.sync_copy(dsts.at[pl.ds(obase, MAXE)], ed)
        pltpu.sync_copy(ws.at[pl.ds(obase, MAXE)], ew)
        nch = jnp.clip((end - obase + CH - 1) // CH, 0, MAXE // CH)

        def issue(ci, par):
            pltpu.async_copy(table.at[es.at[pl.ds(ci * CH, CH)]],
                             rows.at[par], sems[par])

        def wait(par):
            pltpu.make_async_copy(table.at[es.at[pl.ds(0, CH)]],
                                  rows.at[par], sems[par]).wait()

        def compute(ci, par):
            cbase = ci * CH

            def edge16(i, _):
                j0 = i * 16
                g16 = obase + cbase + j0 + lax.iota(jnp.int32, 16)
                valid = jnp.logical_and(g16 >= start, g16 < end)
                w16 = jnp.where(valid, ew[pl.ds(cbase + j0, 16)], 0.0)
                dl16 = jnp.clip(ed[pl.ds(cbase + j0, 16)] - lo, 0, ROWS - 1)
                for l in range(16):
                    wv = jnp.full((16,), w16[l], jnp.float32)
                    dl = dl16[l]
                    vals = [rows[par, j0 + l, pl.ds(q * 16, 16)]
                            for q in range(QD)]
                    prods = [v * wv for v in vals]
                    for q in range(QD):
                        plsc.addupdate(acc.at[dl, pl.ds(q * 16, 16)], prods[q])
                return 0
            lax.fori_loop(0, CH // 16, edge16, 0)

        @pl.when(nch > 0)
        def _():
            issue(0, 0)

        def chunk(ci, _):
            parity = lax.bitwise_and(ci, 1)
            for par in range(2):
                @pl.when(jnp.logical_and(parity == par, ci + 1 < nch))
                def _():
                    issue(ci + 1, 1 - par)

                @pl.when(parity == par)
                def _():
                    wait(par)
                    compute(ci, par)
            return 0
        lax.fori_loop(0, nch, chunk, 0)
        return 0
    lax.fori_loop(0, n_outer, outer, 0)

    for p in range(ROWS // DR):
        r0 = lo + p * DR
        if not first:
            pltpu.sync_copy(prev.at[pl.ds(r0, DR)], prevb)

        def drain(r, _):
            avals = [acc[p * DR + r, pl.ds(q * 16, 16)] for q in range(QD)]
            if first:
                res = avals
            else:
                pvals = [prevb[r, pl.ds(q * 16, 16)] for q in range(QD)]
                res = [2.0 * a - pv for a, pv in zip(avals, pvals)]
            for q in range(QD):
                outb[r, pl.ds(q * 16, 16)] = res[q]
            return 0
        lax.fori_loop(0, DR, drain, 0)
        pltpu.sync_copy(outb, out.at[pl.ds(r0, DR)])


@functools.lru_cache(maxsize=None)
def _make_lap(first, D):
    mesh = plsc.VectorSubcoreMesh(core_axis_name="c", subcore_axis_name="s",
                                  num_cores=NC, num_subcores=NS)
    scratch = [
        pltpu.VMEM((ROWS, D), jnp.float32),
        pltpu.VMEM((2, CH, D), jnp.float32),
        pltpu.VMEM((MAXE,), jnp.int32),
        pltpu.VMEM((MAXE,), jnp.int32),
        pltpu.VMEM((MAXE,), jnp.float32),
        pltpu.VMEM((96,), jnp.int32),
        pltpu.VMEM((DR, D), jnp.float32),
        pltpu.VMEM((DR, D), jnp.float32),
        pltpu.SemaphoreType.DMA,
        pltpu.SemaphoreType.DMA,
    ]
    body = functools.partial(_lap_body, first, D)
    params = None
    if D < 128:
        params = pltpu.CompilerParams(use_tc_tiling_on_sc=False)
    return pl.kernel(body,
                     out_type=jax.ShapeDtypeStruct((NP, D), jnp.float32),
                     mesh=mesh, scratch_types=scratch,
                     compiler_params=params,
                     name=f"sc_lap_{'first' if first else 'rec'}_{D}")


def _lap_first(table, srcs, dsts, ws, se, D):
    dummy = table
    return _make_lap(True, D)(table, srcs, dsts, ws, se, dummy)


def _lap_rec(table, prev, srcs, dsts, ws, se, D):
    return _make_lap(False, D)(table, srcs, dsts, ws, se, prev)


def _cheb_stack(x, srcs, dsts, ws, se, D):
    t0 = x
    t1 = _lap_first(t0, srcs, dsts, ws, se, D)
    t2 = _lap_rec(t1, t0, srcs, dsts, ws, se, D)
    t3 = _lap_rec(t2, t1, srcs, dsts, ws, se, D)
    t4 = _lap_rec(t3, t2, srcs, dsts, ws, se, D)
    return [t0, t1, t2, t3, t4]


BLK = 1024
GRID = NP // BLK


def _dense_step_body(t0_flag, tx0, tx1, tx2, tx3, tx4, th0, th1, th2, th3, th4,
                     c_ref, wx, wh, pb, hn, cn):
    g = jnp.dot(tx0[...], wx[0:F_IN, :], preferred_element_type=jnp.float32)
    for k, txk in enumerate((tx1, tx2, tx3, tx4)):
        g += jnp.dot(txk[...], wx[(k + 1) * F_IN:(k + 2) * F_IN, :],
                     preferred_element_type=jnp.float32)
    if not t0_flag:
        for k, thk in enumerate((th0, th1, th2, th3, th4)):
            g += jnp.dot(thk[...], wh[k * HID:(k + 1) * HID, :],
                         preferred_element_type=jnp.float32)
    w_c_i = pb[0:1, :]
    w_c_f = pb[1:2, :]
    w_c_o = pb[2:3, :]
    b_i = pb[3:4, :]
    b_f = pb[4:5, :]
    b_c = pb[5:6, :]
    b_o = pb[6:7, :]
    c = jnp.zeros((BLK, HID), jnp.float32) if t0_flag else c_ref[...]
    gi = g[:, 0:HID]
    gf = g[:, HID:2 * HID]
    gc = g[:, 2 * HID:3 * HID]
    go = g[:, 3 * HID:4 * HID]
    i_t = jax.nn.sigmoid(gi + w_c_i * c + b_i)
    f_t = jax.nn.sigmoid(gf + w_c_f * c + b_f)
    t_c = jnp.tanh(gc + b_c)
    c_new = f_t * c + i_t * t_c
    o_t = jax.nn.sigmoid(go + w_c_o * c_new + b_o)
    cn[...] = c_new
    hn[...] = o_t * jnp.tanh(c_new)


@functools.lru_cache(maxsize=None)
def _make_dense_step(t0_flag):
    big = pl.BlockSpec((BLK, F_IN), lambda i: (i, 0))
    small = pl.BlockSpec((BLK, HID), lambda i: (i, 0))
    wxs = pl.BlockSpec((K * F_IN, 4 * HID), lambda i: (0, 0))
    whs = pl.BlockSpec((K * HID, 4 * HID), lambda i: (0, 0))
    pbs = pl.BlockSpec((8, HID), lambda i: (0, 0))
    return pl.pallas_call(
        functools.partial(_dense_step_body, t0_flag),
        grid=(GRID,),
        in_specs=[big] * 5 + [small] * 5 + [small, wxs, whs, pbs],
        out_specs=[small, small],
        out_shape=[jax.ShapeDtypeStruct((NP, HID), jnp.float32),
                   jax.ShapeDtypeStruct((NP, HID), jnp.float32)],
        name=f"dense_step_{t0_flag}",
    )


def _head_body(h_ref, wl, bl, out):
    out[...] = jnp.dot(jax.nn.relu(h_ref[...]), wl[...],
                       preferred_element_type=jnp.float32) + bl[...]


@functools.lru_cache(maxsize=None)
def _make_head():
    return pl.pallas_call(
        _head_body,
        grid=(GRID,),
        in_specs=[pl.BlockSpec((BLK, HID), lambda i: (i, 0)),
                  pl.BlockSpec((HID, 128), lambda i: (0, 0)),
                  pl.BlockSpec((1, 128), lambda i: (0, 0))],
        out_specs=pl.BlockSpec((BLK, 128), lambda i: (i, 0)),
        out_shape=jax.ShapeDtypeStruct((NP, 128), jnp.float32),
        name="head",
    )


def kernel(timesteps, edge_index, Wx_i, Wh_i, w_c_i, b_i, Wx_f, Wh_f, w_c_f, b_f,
           Wx_c, Wh_c, b_c, Wx_o, Wh_o, w_c_o, b_o, W_lin, b_lin):
    src = edge_index[0]
    dst = edge_index[1]
    mask = src != dst
    deg = jax.ops.segment_sum(mask.astype(jnp.float32), src, num_segments=N)
    dis = jnp.where(deg > 0, 1.0 / jnp.sqrt(jnp.maximum(deg, 1e-12)), 0.0)
    w = jnp.where(mask, -dis[src] * dis[dst], 0.0)

    order = jnp.argsort(dst)
    srcs = jnp.pad(src[order].astype(jnp.int32), (0, MAXE))
    dsts = jnp.pad(dst[order].astype(jnp.int32), (0, MAXE))
    ws = jnp.pad(w[order], (0, MAXE))
    bounds = jnp.arange(NW, dtype=jnp.int32) * ROWS
    starts = jnp.searchsorted(dsts, bounds, side="left").astype(jnp.int32)
    ends = jnp.searchsorted(dsts, bounds + ROWS, side="left").astype(jnp.int32)
    se = jnp.concatenate([starts, ends, jnp.zeros((32,), jnp.int32)])

    WxAll = jnp.concatenate([Wx_i, Wx_f, Wx_c, Wx_o], axis=2).reshape(K * F_IN, 4 * HID)
    WhAll = jnp.concatenate([Wh_i, Wh_f, Wh_c, Wh_o], axis=2).reshape(K * HID, 4 * HID)
    PB = jnp.concatenate([w_c_i, w_c_f, w_c_o, b_i, b_f, b_c, b_o,
                          jnp.zeros((1, HID), jnp.float32)], axis=0)

    xp = jnp.pad(timesteps, ((0, 0), (0, NP - N), (0, 0)))
    W_lin_p = jnp.pad(W_lin, ((0, 0), (0, 128 - W_lin.shape[1])))
    b_lin_p = jnp.pad(b_lin, ((0, 128 - b_lin.shape[0]),)).reshape(1, 128)

    zeros_h = jnp.zeros((NP, HID), jnp.float32)
    H = zeros_h
    C = zeros_h
    for t in range(T):
        txs = _cheb_stack(xp[t], srcs, dsts, ws, se, F_IN)
        if t == 0:
            ths = [zeros_h] * K
        else:
            ths = _cheb_stack(H, srcs, dsts, ws, se, HID)
        H, C = _make_dense_step(t == 0)(*txs, *ths, C, WxAll, WhAll, PB)

    out = _make_head()(H, W_lin_p, b_lin_p)
    return out[:N, :T]


# speedup vs baseline: 4.6813x; 1.4586x over previous
import functools

import jax
import jax.numpy as jnp
from jax import lax
from jax.experimental import pallas as pl
from jax.experimental.pallas import tpu as pltpu
from jax.experimental.pallas import tpu_sc as plsc

N = 10000
E = 320000
F_IN = 128
HID = 32
K = 5
T = 12

NC = 2
NS = 16
NW = NC * NS
ROWS = 320
NP = NW * ROWS
CH = 128
MAXE = 4096
NB = 4
DR = 80


def _lap_body(first, D, table, srcs, dsts, ws, se, prev, out,
              acc, rows, es, ed, ew, sev, prevb,
              sem0, sem1, sem2, sem3):
    QD = D // 16
    wid = lax.axis_index("s") * NC + lax.axis_index("c")
    lo = wid * ROWS

    pltpu.sync_copy(se, sev)
    start = sev[pl.ds(wid, 16)][0]
    end = sev[pl.ds(NW + wid, 16)][0]

    start8 = (start // 8) * 8
    n_outer = (end - start8 + MAXE - 1) // MAXE
    sems = (sem0, sem1, sem2, sem3)

    def zrow(r, _):
        for q in range(QD):
            acc[r, pl.ds(q * 16, 16)] = jnp.zeros((16,), jnp.float32)
        return 0
    lax.fori_loop(0, ROWS, zrow, 0)

    def outer(o, _):
        obase = start8 + o * MAXE
        pltpu.sync_copy(srcs.at[pl.ds(obase, MAXE)], es)
        pltpu.sync_copy(dsts.at[pl.ds(obase, MAXE)], ed)
        pltpu.sync_copy(ws.at[pl.ds(obase, MAXE)], ew)
        nch = jnp.clip((end - obase + CH - 1) // CH, 0, MAXE // CH)

        def issue(ci, slot):
            pltpu.async_copy(table.at[es.at[pl.ds(ci * CH, CH)]],
                             rows.at[slot], sems[slot])

        def wait(slot):
            pltpu.make_async_copy(table.at[es.at[pl.ds(0, CH)]],
                                  rows.at[slot], sems[slot]).wait()

        def compute(ci, slot):
            cbase = ci * CH

            def edge16(i, _):
                j0 = i * 16
                g16 = obase + cbase + j0 + lax.iota(jnp.int32, 16)
                valid = jnp.logical_and(g16 >= start, g16 < end)
                w16 = jnp.where(valid, ew[pl.ds(cbase + j0, 16)], 0.0)
                dl16 = jnp.clip(ed[pl.ds(cbase + j0, 16)] - lo, 0, ROWS - 1)
                for l in range(16):
                    wv = jnp.full((16,), w16[l], jnp.float32)
                    dl = dl16[l]
                    vals = [rows[slot, j0 + l, pl.ds(q * 16, 16)]
                            for q in range(QD)]
                    prods = [v * wv for v in vals]
                    for q in range(QD):
                        plsc.addupdate(acc.at[dl, pl.ds(q * 16, 16)], prods[q])
                return 0
            lax.fori_loop(0, CH // 16, edge16, 0)

        for k in range(NB):
            @pl.when(k < nch)
            def _():
                issue(k, k)

        def chunk(ci, _):
            slotv = lax.bitwise_and(ci, NB - 1)
            for slot in range(NB):
                @pl.when(slotv == slot)
                def _():
                    wait(slot)
                    compute(ci, slot)

                @pl.when(jnp.logical_and(slotv == slot, ci + NB < nch))
                def _():
                    issue(ci + NB, slot)
            return 0
        lax.fori_loop(0, nch, chunk, 0)
        return 0
    lax.fori_loop(0, n_outer, outer, 0)

    if not first:
        for p in range(ROWS // DR):
            pltpu.sync_copy(prev.at[pl.ds(lo + p * DR, DR)], prevb)

            def drain(r, _):
                avals = [acc[p * DR + r, pl.ds(q * 16, 16)] for q in range(QD)]
                pvals = [prevb[r, pl.ds(q * 16, 16)] for q in range(QD)]
                res = [2.0 * a - pv for a, pv in zip(avals, pvals)]
                for q in range(QD):
                    acc[p * DR + r, pl.ds(q * 16, 16)] = res[q]
                return 0
            lax.fori_loop(0, DR, drain, 0)
    pltpu.sync_copy(acc, out.at[pl.ds(lo, ROWS)])


@functools.lru_cache(maxsize=None)
def _make_lap(first, D):
    mesh = plsc.VectorSubcoreMesh(core_axis_name="c", subcore_axis_name="s",
                                  num_cores=NC, num_subcores=NS)
    scratch = [
        pltpu.VMEM((ROWS, D), jnp.float32),
        pltpu.VMEM((NB, CH, D), jnp.float32),
        pltpu.VMEM((MAXE,), jnp.int32),
        pltpu.VMEM((MAXE,), jnp.int32),
        pltpu.VMEM((MAXE,), jnp.float32),
        pltpu.VMEM((96,), jnp.int32),
        pltpu.VMEM((DR, D), jnp.float32),
        pltpu.SemaphoreType.DMA,
        pltpu.SemaphoreType.DMA,
        pltpu.SemaphoreType.DMA,
        pltpu.SemaphoreType.DMA,
    ]
    body = functools.partial(_lap_body, first, D)
    params = None
    if D < 128:
        params = pltpu.CompilerParams(use_tc_tiling_on_sc=False)
    return pl.kernel(body,
                     out_type=jax.ShapeDtypeStruct((NP, D), jnp.float32),
                     mesh=mesh, scratch_types=scratch,
                     compiler_params=params,
                     name=f"sc_lap_{'first' if first else 'rec'}_{D}")


def _lap_first(table, srcs, dsts, ws, se, D):
    dummy = table
    return _make_lap(True, D)(table, srcs, dsts, ws, se, dummy)


def _lap_rec(table, prev, srcs, dsts, ws, se, D):
    return _make_lap(False, D)(table, srcs, dsts, ws, se, prev)


def _cheb_stack(x, srcs, dsts, ws, se, D):
    t0 = x
    t1 = _lap_first(t0, srcs, dsts, ws, se, D)
    t2 = _lap_rec(t1, t0, srcs, dsts, ws, se, D)
    t3 = _lap_rec(t2, t1, srcs, dsts, ws, se, D)
    t4 = _lap_rec(t3, t2, srcs, dsts, ws, se, D)
    return [t0, t1, t2, t3, t4]


BLK = 1024
GRID = NP // BLK


def _dense_step_body(t0_flag, tx0, tx1, tx2, tx3, tx4, th0, th1, th2, th3, th4,
                     c_ref, wx, wh, pb, hn, cn):
    g = jnp.dot(tx0[...], wx[0:F_IN, :], preferred_element_type=jnp.float32)
    for k, txk in enumerate((tx1, tx2, tx3, tx4)):
        g += jnp.dot(txk[...], wx[(k + 1) * F_IN:(k + 2) * F_IN, :],
                     preferred_element_type=jnp.float32)
    if not t0_flag:
        for k, thk in enumerate((th0, th1, th2, th3, th4)):
            g += jnp.dot(thk[...], wh[k * HID:(k + 1) * HID, :],
                         preferred_element_type=jnp.float32)
    w_c_i = pb[0:1, :]
    w_c_f = pb[1:2, :]
    w_c_o = pb[2:3, :]
    b_i = pb[3:4, :]
    b_f = pb[4:5, :]
    b_c = pb[5:6, :]
    b_o = pb[6:7, :]
    c = jnp.zeros((BLK, HID), jnp.float32) if t0_flag else c_ref[...]
    gi = g[:, 0:HID]
    gf = g[:, HID:2 * HID]
    gc = g[:, 2 * HID:3 * HID]
    go = g[:, 3 * HID:4 * HID]
    i_t = jax.nn.sigmoid(gi + w_c_i * c + b_i)
    f_t = jax.nn.sigmoid(gf + w_c_f * c + b_f)
    t_c = jnp.tanh(gc + b_c)
    c_new = f_t * c + i_t * t_c
    o_t = jax.nn.sigmoid(go + w_c_o * c_new + b_o)
    cn[...] = c_new
    hn[...] = o_t * jnp.tanh(c_new)


@functools.lru_cache(maxsize=None)
def _make_dense_step(t0_flag):
    big = pl.BlockSpec((BLK, F_IN), lambda i: (i, 0))
    small = pl.BlockSpec((BLK, HID), lambda i: (i, 0))
    wxs = pl.BlockSpec((K * F_IN, 4 * HID), lambda i: (0, 0))
    whs = pl.BlockSpec((K * HID, 4 * HID), lambda i: (0, 0))
    pbs = pl.BlockSpec((8, HID), lambda i: (0, 0))
    return pl.pallas_call(
        functools.partial(_dense_step_body, t0_flag),
        grid=(GRID,),
        in_specs=[big] * 5 + [small] * 5 + [small, wxs, whs, pbs],
        out_specs=[small, small],
        out_shape=[jax.ShapeDtypeStruct((NP, HID), jnp.float32),
                   jax.ShapeDtypeStruct((NP, HID), jnp.float32)],
        name=f"dense_step_{t0_flag}",
    )


def _head_body(h_ref, wl, bl, out):
    out[...] = jnp.dot(jax.nn.relu(h_ref[...]), wl[...],
                       preferred_element_type=jnp.float32) + bl[...]


@functools.lru_cache(maxsize=None)
def _make_head():
    return pl.pallas_call(
        _head_body,
        grid=(GRID,),
        in_specs=[pl.BlockSpec((BLK, HID), lambda i: (i, 0)),
                  pl.BlockSpec((HID, 128), lambda i: (0, 0)),
                  pl.BlockSpec((1, 128), lambda i: (0, 0))],
        out_specs=pl.BlockSpec((BLK, 128), lambda i: (i, 0)),
        out_shape=jax.ShapeDtypeStruct((NP, 128), jnp.float32),
        name="head",
    )


def kernel(timesteps, edge_index, Wx_i, Wh_i, w_c_i, b_i, Wx_f, Wh_f, w_c_f, b_f,
           Wx_c, Wh_c, b_c, Wx_o, Wh_o, w_c_o, b_o, W_lin, b_lin):
    src = edge_index[0]
    dst = edge_index[1]
    mask = src != dst
    deg = jax.ops.segment_sum(mask.astype(jnp.float32), src, num_segments=N)
    dis = jnp.where(deg > 0, 1.0 / jnp.sqrt(jnp.maximum(deg, 1e-12)), 0.0)
    w = jnp.where(mask, -dis[src] * dis[dst], 0.0)

    order = jnp.argsort(dst)
    srcs = jnp.pad(src[order].astype(jnp.int32), (0, MAXE))
    dsts = jnp.pad(dst[order].astype(jnp.int32), (0, MAXE))
    ws = jnp.pad(w[order], (0, MAXE))
    bounds = jnp.arange(NW, dtype=jnp.int32) * ROWS
    starts = jnp.searchsorted(dsts, bounds, side="left").astype(jnp.int32)
    ends = jnp.searchsorted(dsts, bounds + ROWS, side="left").astype(jnp.int32)
    se = jnp.concatenate([starts, ends, jnp.zeros((32,), jnp.int32)])

    WxAll = jnp.concatenate([Wx_i, Wx_f, Wx_c, Wx_o], axis=2).reshape(K * F_IN, 4 * HID)
    WhAll = jnp.concatenate([Wh_i, Wh_f, Wh_c, Wh_o], axis=2).reshape(K * HID, 4 * HID)
    PB = jnp.concatenate([w_c_i, w_c_f, w_c_o, b_i, b_f, b_c, b_o,
                          jnp.zeros((1, HID), jnp.float32)], axis=0)

    xp = jnp.pad(timesteps, ((0, 0), (0, NP - N), (0, 0)))
    W_lin_p = jnp.pad(W_lin, ((0, 0), (0, 128 - W_lin.shape[1])))
    b_lin_p = jnp.pad(b_lin, ((0, 128 - b_lin.shape[0]),)).reshape(1, 128)

    zeros_h = jnp.zeros((NP, HID), jnp.float32)
    H = zeros_h
    C = zeros_h
    for t in range(T):
        txs = _cheb_stack(xp[t], srcs, dsts, ws, se, F_IN)
        if t == 0:
            ths = [zeros_h] * K
        else:
            ths = _cheb_stack(H, srcs, dsts, ws, se, HID)
        H, C = _make_dense_step(t == 0)(*txs, *ths, C, WxAll, WhAll, PB)

    out = _make_head()(H, W_lin_p, b_lin_p)
    return out[:N, :T]


# speedup vs baseline: 5.5526x; 1.1861x over previous
import functools

import jax
import jax.numpy as jnp
from jax import lax
from jax.experimental import pallas as pl
from jax.experimental.pallas import tpu as pltpu
from jax.experimental.pallas import tpu_sc as plsc

N = 10000
E = 320000
F_IN = 128
HID = 32
K = 5
T = 12

NC = 2
NS = 16
NW = NC * NS
ROWS = 320
NP = NW * ROWS
CH = 128
MAXE = 4096
NB = 2
DR = 80


def _lap_body(first, tableX, tableH, srcs, dsts, ws, se, prevX, prevH,
              outX, outH, accX, accH, rowsX, rowsH, es, ed, ew, sev,
              prevbX, prevbH, semX0, semX1, semH0, semH1):
    QX = F_IN // 16
    QH = HID // 16
    wid = lax.axis_index("s") * NC + lax.axis_index("c")
    lo = wid * ROWS

    pltpu.sync_copy(se, sev)
    start = sev[pl.ds(wid, 16)][0]
    end = sev[pl.ds(NW + wid, 16)][0]

    start8 = (start // 8) * 8
    n_outer = (end - start8 + MAXE - 1) // MAXE
    semsX = (semX0, semX1)
    semsH = (semH0, semH1)

    def zrow(r, _):
        for q in range(QX):
            accX[r, pl.ds(q * 16, 16)] = jnp.zeros((16,), jnp.float32)
        for q in range(QH):
            accH[r, pl.ds(q * 16, 16)] = jnp.zeros((16,), jnp.float32)
        return 0
    lax.fori_loop(0, ROWS, zrow, 0)

    def outer(o, _):
        obase = start8 + o * MAXE
        pltpu.sync_copy(srcs.at[pl.ds(obase, MAXE)], es)
        pltpu.sync_copy(dsts.at[pl.ds(obase, MAXE)], ed)
        pltpu.sync_copy(ws.at[pl.ds(obase, MAXE)], ew)
        nch = jnp.clip((end - obase + CH - 1) // CH, 0, MAXE // CH)

        def issue(ci, slot):
            idx = es.at[pl.ds(ci * CH, CH)]
            pltpu.async_copy(tableX.at[idx], rowsX.at[slot], semsX[slot])
            pltpu.async_copy(tableH.at[idx], rowsH.at[slot], semsH[slot])

        def wait(slot):
            idx = es.at[pl.ds(0, CH)]
            pltpu.make_async_copy(tableX.at[idx], rowsX.at[slot],
                                  semsX[slot]).wait()
            pltpu.make_async_copy(tableH.at[idx], rowsH.at[slot],
                                  semsH[slot]).wait()

        def compute(ci, slot):
            cbase = ci * CH

            def edge16(i, _):
                j0 = i * 16
                g16 = obase + cbase + j0 + lax.iota(jnp.int32, 16)
                valid = jnp.logical_and(g16 >= start, g16 < end)
                w16 = jnp.where(valid, ew[pl.ds(cbase + j0, 16)], 0.0)
                dl16 = jnp.clip(ed[pl.ds(cbase + j0, 16)] - lo, 0, ROWS - 1)
                for l in range(16):
                    wv = jnp.full((16,), w16[l], jnp.float32)
                    dl = dl16[l]
                    valsX = [rowsX[slot, j0 + l, pl.ds(q * 16, 16)]
                             for q in range(QX)]
                    valsH = [rowsH[slot, j0 + l, pl.ds(q * 16, 16)]
                             for q in range(QH)]
                    prodsX = [v * wv for v in valsX]
                    prodsH = [v * wv for v in valsH]
                    for q in range(QX):
                        plsc.addupdate(accX.at[dl, pl.ds(q * 16, 16)],
                                       prodsX[q])
                    for q in range(QH):
                        plsc.addupdate(accH.at[dl, pl.ds(q * 16, 16)],
                                       prodsH[q])
                return 0
            lax.fori_loop(0, CH // 16, edge16, 0)

        for k in range(NB):
            @pl.when(k < nch)
            def _():
                issue(k, k)

        def chunk(ci, _):
            slotv = lax.bitwise_and(ci, NB - 1)
            for slot in range(NB):
                @pl.when(slotv == slot)
                def _():
                    wait(slot)
                    compute(ci, slot)

                @pl.when(jnp.logical_and(slotv == slot, ci + NB < nch))
                def _():
                    issue(ci + NB, slot)
            return 0
        lax.fori_loop(0, nch, chunk, 0)
        return 0
    lax.fori_loop(0, n_outer, outer, 0)

    if not first:
        for p in range(ROWS // DR):
            pltpu.sync_copy(prevX.at[pl.ds(lo + p * DR, DR)], prevbX)
            pltpu.sync_copy(prevH.at[pl.ds(lo + p * DR, DR)], prevbH)

            def drain(r, _):
                for q in range(QX):
                    a = accX[p * DR + r, pl.ds(q * 16, 16)]
                    pv = prevbX[r, pl.ds(q * 16, 16)]
                    accX[p * DR + r, pl.ds(q * 16, 16)] = 2.0 * a - pv
                for q in range(QH):
                    a = accH[p * DR + r, pl.ds(q * 16, 16)]
                    pv = prevbH[r, pl.ds(q * 16, 16)]
                    accH[p * DR + r, pl.ds(q * 16, 16)] = 2.0 * a - pv
                return 0
            lax.fori_loop(0, DR, drain, 0)
    pltpu.sync_copy(accX, outX.at[pl.ds(lo, ROWS)])
    pltpu.sync_copy(accH, outH.at[pl.ds(lo, ROWS)])


@functools.lru_cache(maxsize=None)
def _make_lap(first):
    mesh = plsc.VectorSubcoreMesh(core_axis_name="c", subcore_axis_name="s",
                                  num_cores=NC, num_subcores=NS)
    scratch = [
        pltpu.VMEM((ROWS, F_IN), jnp.float32),
        pltpu.VMEM((ROWS, HID), jnp.float32),
        pltpu.VMEM((NB, CH, F_IN), jnp.float32),
        pltpu.VMEM((NB, CH, HID), jnp.float32),
        pltpu.VMEM((MAXE,), jnp.int32),
        pltpu.VMEM((MAXE,), jnp.int32),
        pltpu.VMEM((MAXE,), jnp.float32),
        pltpu.VMEM((96,), jnp.int32),
        pltpu.VMEM((DR, F_IN), jnp.float32),
        pltpu.VMEM((DR, HID), jnp.float32),
        pltpu.SemaphoreType.DMA,
        pltpu.SemaphoreType.DMA,
        pltpu.SemaphoreType.DMA,
        pltpu.SemaphoreType.DMA,
    ]
    body = functools.partial(_lap_body, first)
    params = pltpu.CompilerParams(use_tc_tiling_on_sc=False)
    return pl.kernel(body,
                     out_type=(jax.ShapeDtypeStruct((NP, F_IN), jnp.float32),
                               jax.ShapeDtypeStruct((NP, HID), jnp.float32)),
                     mesh=mesh, scratch_types=scratch,
                     compiler_params=params,
                     name=f"sc_lap2_{'first' if first else 'rec'}")


def _cheb2_stack(x, h, srcs, dsts, ws, se):
    tx = [x]
    th = [h]
    t1x, t1h = _make_lap(True)(x, h, srcs, dsts, ws, se, x, h)
    tx.append(t1x)
    th.append(t1h)
    for _ in range(2, K):
        tnx, tnh = _make_lap(False)(tx[-1], th[-1], srcs, dsts, ws, se,
                                    tx[-2], th[-2])
        tx.append(tnx)
        th.append(tnh)
    return tx, th


BLK = 1024
GRID = NP // BLK


def _dense_step_body(t0_flag, tx0, tx1, tx2, tx3, tx4, th0, th1, th2, th3, th4,
                     c_ref, wx, wh, pb, hn, cn):
    g = jnp.dot(tx0[...], wx[0:F_IN, :], preferred_element_type=jnp.float32)
    for k, txk in enumerate((tx1, tx2, tx3, tx4)):
        g += jnp.dot(txk[...], wx[(k + 1) * F_IN:(k + 2) * F_IN, :],
                     preferred_element_type=jnp.float32)
    if not t0_flag:
        for k, thk in enumerate((th0, th1, th2, th3, th4)):
            g += jnp.dot(thk[...], wh[k * HID:(k + 1) * HID, :],
                         preferred_element_type=jnp.float32)
    w_c_i = pb[0:1, :]
    w_c_f = pb[1:2, :]
    w_c_o = pb[2:3, :]
    b_i = pb[3:4, :]
    b_f = pb[4:5, :]
    b_c = pb[5:6, :]
    b_o = pb[6:7, :]
    c = jnp.zeros((BLK, HID), jnp.float32) if t0_flag else c_ref[...]
    gi = g[:, 0:HID]
    gf = g[:, HID:2 * HID]
    gc = g[:, 2 * HID:3 * HID]
    go = g[:, 3 * HID:4 * HID]
    i_t = jax.nn.sigmoid(gi + w_c_i * c + b_i)
    f_t = jax.nn.sigmoid(gf + w_c_f * c + b_f)
    t_c = jnp.tanh(gc + b_c)
    c_new = f_t * c + i_t * t_c
    o_t = jax.nn.sigmoid(go + w_c_o * c_new + b_o)
    cn[...] = c_new
    hn[...] = o_t * jnp.tanh(c_new)


@functools.lru_cache(maxsize=None)
def _make_dense_step(t0_flag):
    big = pl.BlockSpec((BLK, F_IN), lambda i: (i, 0))
    small = pl.BlockSpec((BLK, HID), lambda i: (i, 0))
    wxs = pl.BlockSpec((K * F_IN, 4 * HID), lambda i: (0, 0))
    whs = pl.BlockSpec((K * HID, 4 * HID), lambda i: (0, 0))
    pbs = pl.BlockSpec((8, HID), lambda i: (0, 0))
    return pl.pallas_call(
        functools.partial(_dense_step_body, t0_flag),
        grid=(GRID,),
        in_specs=[big] * 5 + [small] * 5 + [small, wxs, whs, pbs],
        out_specs=[small, small],
        out_shape=[jax.ShapeDtypeStruct((NP, HID), jnp.float32),
                   jax.ShapeDtypeStruct((NP, HID), jnp.float32)],
        name=f"dense_step_{t0_flag}",
    )


def _head_body(h_ref, wl, bl, out):
    out[...] = jnp.dot(jax.nn.relu(h_ref[...]), wl[...],
                       preferred_element_type=jnp.float32) + bl[...]


@functools.lru_cache(maxsize=None)
def _make_head():
    return pl.pallas_call(
        _head_body,
        grid=(GRID,),
        in_specs=[pl.BlockSpec((BLK, HID), lambda i: (i, 0)),
                  pl.BlockSpec((HID, 128), lambda i: (0, 0)),
                  pl.BlockSpec((1, 128), lambda i: (0, 0))],
        out_specs=pl.BlockSpec((BLK, 128), lambda i: (i, 0)),
        out_shape=jax.ShapeDtypeStruct((NP, 128), jnp.float32),
        name="head",
    )


def kernel(timesteps, edge_index, Wx_i, Wh_i, w_c_i, b_i, Wx_f, Wh_f, w_c_f, b_f,
           Wx_c, Wh_c, b_c, Wx_o, Wh_o, w_c_o, b_o, W_lin, b_lin):
    src = edge_index[0]
    dst = edge_index[1]
    mask = src != dst
    deg = jax.ops.segment_sum(mask.astype(jnp.float32), src, num_segments=N)
    dis = jnp.where(deg > 0, 1.0 / jnp.sqrt(jnp.maximum(deg, 1e-12)), 0.0)
    w = jnp.where(mask, -dis[src] * dis[dst], 0.0)

    order = jnp.argsort(dst)
    srcs = jnp.pad(src[order].astype(jnp.int32), (0, MAXE))
    dsts = jnp.pad(dst[order].astype(jnp.int32), (0, MAXE))
    ws = jnp.pad(w[order], (0, MAXE))
    bounds = jnp.arange(NW, dtype=jnp.int32) * ROWS
    starts = jnp.searchsorted(dsts, bounds, side="left").astype(jnp.int32)
    ends = jnp.searchsorted(dsts, bounds + ROWS, side="left").astype(jnp.int32)
    se = jnp.concatenate([starts, ends, jnp.zeros((32,), jnp.int32)])

    WxAll = jnp.concatenate([Wx_i, Wx_f, Wx_c, Wx_o], axis=2).reshape(K * F_IN, 4 * HID)
    WhAll = jnp.concatenate([Wh_i, Wh_f, Wh_c, Wh_o], axis=2).reshape(K * HID, 4 * HID)
    PB = jnp.concatenate([w_c_i, w_c_f, w_c_o, b_i, b_f, b_c, b_o,
                          jnp.zeros((1, HID), jnp.float32)], axis=0)

    xp = jnp.pad(timesteps, ((0, 0), (0, NP - N), (0, 0)))
    W_lin_p = jnp.pad(W_lin, ((0, 0), (0, 128 - W_lin.shape[1])))
    b_lin_p = jnp.pad(b_lin, ((0, 128 - b_lin.shape[0]),)).reshape(1, 128)

    zeros_h = jnp.zeros((NP, HID), jnp.float32)
    H = zeros_h
    C = zeros_h
    for t in range(T):
        txs, ths = _cheb2_stack(xp[t], H, srcs, dsts, ws, se)
        H, C = _make_dense_step(t == 0)(*txs, *ths, C, WxAll, WhAll, PB)

    out = _make_head()(H, W_lin_p, b_lin_p)
    return out[:N, :T]


# speedup vs baseline: 6.4455x; 1.1608x over previous
import functools

import jax
import jax.numpy as jnp
from jax import lax
from jax.experimental import pallas as pl
from jax.experimental.pallas import tpu as pltpu
from jax.experimental.pallas import tpu_sc as plsc

N = 10000
E = 320000
F_IN = 128
HID = 32
K = 5
T = 12

NC = 2
NS = 16
NW = NC * NS
ROWS = 320
NP = NW * ROWS
CH = 128
MAXE = 2304
NB = 3
DR = 64


def _lap_body(first, tableX, tableH, srcs, dsts, ws, se, prevX, prevH,
              outX, outH, accX, accH, rowsX, rowsH, es, ed, ew, sev,
              prevbX, prevbH, semX0, semX1, semX2, semH0, semH1, semH2):
    QX = F_IN // 16
    QH = HID // 16
    wid = lax.axis_index("s") * NC + lax.axis_index("c")
    lo = wid * ROWS

    pltpu.sync_copy(se, sev)
    start = sev[pl.ds(wid, 16)][0]
    end = sev[pl.ds(NW + wid, 16)][0]

    start8 = (start // 8) * 8
    n_outer = (end - start8 + MAXE - 1) // MAXE
    semsX = (semX0, semX1, semX2)
    semsH = (semH0, semH1, semH2)

    def zrow(r, _):
        for q in range(QX):
            accX[r, pl.ds(q * 16, 16)] = jnp.zeros((16,), jnp.float32)
        for q in range(QH):
            accH[r, pl.ds(q * 16, 16)] = jnp.zeros((16,), jnp.float32)
        return 0
    lax.fori_loop(0, ROWS, zrow, 0)

    def outer(o, _):
        obase = start8 + o * MAXE
        pltpu.sync_copy(srcs.at[pl.ds(obase, MAXE)], es)
        pltpu.sync_copy(dsts.at[pl.ds(obase, MAXE)], ed)
        pltpu.sync_copy(ws.at[pl.ds(obase, MAXE)], ew)
        nch = jnp.clip((end - obase + CH - 1) // CH, 0, MAXE // CH)

        def issue(ci, slot):
            idx = es.at[pl.ds(ci * CH, CH)]
            pltpu.async_copy(tableX.at[idx], rowsX.at[slot], semsX[slot])
            pltpu.async_copy(tableH.at[idx], rowsH.at[slot], semsH[slot])

        def wait(slot):
            idx = es.at[pl.ds(0, CH)]
            pltpu.make_async_copy(tableX.at[idx], rowsX.at[slot],
                                  semsX[slot]).wait()
            pltpu.make_async_copy(tableH.at[idx], rowsH.at[slot],
                                  semsH[slot]).wait()

        def compute(ci, slot):
            cbase = ci * CH

            def edge16(i, _):
                j0 = i * 16
                g16 = obase + cbase + j0 + lax.iota(jnp.int32, 16)
                valid = jnp.logical_and(g16 >= start, g16 < end)
                w16 = jnp.where(valid, ew[pl.ds(cbase + j0, 16)], 0.0)
                dl16 = jnp.clip(ed[pl.ds(cbase + j0, 16)] - lo, 0, ROWS - 1)
                for l in range(16):
                    wv = jnp.full((16,), w16[l], jnp.float32)
                    dl = dl16[l]
                    valsX = [rowsX[slot, j0 + l, pl.ds(q * 16, 16)]
                             for q in range(QX)]
                    valsH = [rowsH[slot, j0 + l, pl.ds(q * 16, 16)]
                             for q in range(QH)]
                    prodsX = [v * wv for v in valsX]
                    prodsH = [v * wv for v in valsH]
                    for q in range(QX):
                        plsc.addupdate(accX.at[dl, pl.ds(q * 16, 16)],
                                       prodsX[q])
                    for q in range(QH):
                        plsc.addupdate(accH.at[dl, pl.ds(q * 16, 16)],
                                       prodsH[q])
                return 0
            lax.fori_loop(0, CH // 16, edge16, 0)

        for k in range(NB):
            @pl.when(k < nch)
            def _():
                issue(k, k)

        def chunk(ci, slotv):
            for slot in range(NB):
                @pl.when(slotv == slot)
                def _():
                    wait(slot)
                    compute(ci, slot)

                @pl.when(jnp.logical_and(slotv == slot, ci + NB < nch))
                def _():
                    issue(ci + NB, slot)
            return jnp.where(slotv == NB - 1, 0, slotv + 1)
        lax.fori_loop(0, nch, chunk, jnp.int32(0))
        return 0
    lax.fori_loop(0, n_outer, outer, 0)

    if not first:
        for p in range(ROWS // DR):
            pltpu.sync_copy(prevX.at[pl.ds(lo + p * DR, DR)], prevbX)
            pltpu.sync_copy(prevH.at[pl.ds(lo + p * DR, DR)], prevbH)

            def drain(r, _):
                for q in range(QX):
                    a = accX[p * DR + r, pl.ds(q * 16, 16)]
                    pv = prevbX[r, pl.ds(q * 16, 16)]
                    accX[p * DR + r, pl.ds(q * 16, 16)] = 2.0 * a - pv
                for q in range(QH):
                    a = accH[p * DR + r, pl.ds(q * 16, 16)]
                    pv = prevbH[r, pl.ds(q * 16, 16)]
                    accH[p * DR + r, pl.ds(q * 16, 16)] = 2.0 * a - pv
                return 0
            lax.fori_loop(0, DR, drain, 0)
    pltpu.sync_copy(accX, outX.at[pl.ds(lo, ROWS)])
    pltpu.sync_copy(accH, outH.at[pl.ds(lo, ROWS)])


@functools.lru_cache(maxsize=None)
def _make_lap(first):
    mesh = plsc.VectorSubcoreMesh(core_axis_name="c", subcore_axis_name="s",
                                  num_cores=NC, num_subcores=NS)
    scratch = [
        pltpu.VMEM((ROWS, F_IN), jnp.float32),
        pltpu.VMEM((ROWS, HID), jnp.float32),
        pltpu.VMEM((NB, CH, F_IN), jnp.float32),
        pltpu.VMEM((NB, CH, HID), jnp.float32),
        pltpu.VMEM((MAXE,), jnp.int32),
        pltpu.VMEM((MAXE,), jnp.int32),
        pltpu.VMEM((MAXE,), jnp.float32),
        pltpu.VMEM((96,), jnp.int32),
        pltpu.VMEM((DR, F_IN), jnp.float32),
        pltpu.VMEM((DR, HID), jnp.float32),
        pltpu.SemaphoreType.DMA,
        pltpu.SemaphoreType.DMA,
        pltpu.SemaphoreType.DMA,
        pltpu.SemaphoreType.DMA,
        pltpu.SemaphoreType.DMA,
        pltpu.SemaphoreType.DMA,
    ]
    body = functools.partial(_lap_body, first)
    params = pltpu.CompilerParams(use_tc_tiling_on_sc=False)
    return pl.kernel(body,
                     out_type=(jax.ShapeDtypeStruct((NP, F_IN), jnp.float32),
                               jax.ShapeDtypeStruct((NP, HID), jnp.float32)),
                     mesh=mesh, scratch_types=scratch,
                     compiler_params=params,
                     name=f"sc_lap2_{'first' if first else 'rec'}")


def _cheb2_stack(x, h, srcs, dsts, ws, se):
    tx = [x]
    th = [h]
    t1x, t1h = _make_lap(True)(x, h, srcs, dsts, ws, se, x, h)
    tx.append(t1x)
    th.append(t1h)
    for _ in range(2, K):
        tnx, tnh = _make_lap(False)(tx[-1], th[-1], srcs, dsts, ws, se,
                                    tx[-2], th[-2])
        tx.append(tnx)
        th.append(tnh)
    return tx, th


BLK = 1024
GRID = NP // BLK


def _dense_step_body(t0_flag, tx0, tx1, tx2, tx3, tx4, th0, th1, th2, th3, th4,
                     c_ref, wx, wh, pb, hn, cn):
    g = jnp.dot(tx0[...], wx[0:F_IN, :], preferred_element_type=jnp.float32)
    for k, txk in enumerate((tx1, tx2, tx3, tx4)):
        g += jnp.dot(txk[...], wx[(k + 1) * F_IN:(k + 2) * F_IN, :],
                     preferred_element_type=jnp.float32)
    if not t0_flag:
        for k, thk in enumerate((th0, th1, th2, th3, th4)):
            g += jnp.dot(thk[...], wh[k * HID:(k + 1) * HID, :],
                         preferred_element_type=jnp.float32)
    w_c_i = pb[0:1, :]
    w_c_f = pb[1:2, :]
    w_c_o = pb[2:3, :]
    b_i = pb[3:4, :]
    b_f = pb[4:5, :]
    b_c = pb[5:6, :]
    b_o = pb[6:7, :]
    c = jnp.zeros((BLK, HID), jnp.float32) if t0_flag else c_ref[...]
    gi = g[:, 0:HID]
    gf = g[:, HID:2 * HID]
    gc = g[:, 2 * HID:3 * HID]
    go = g[:, 3 * HID:4 * HID]
    i_t = jax.nn.sigmoid(gi + w_c_i * c + b_i)
    f_t = jax.nn.sigmoid(gf + w_c_f * c + b_f)
    t_c = jnp.tanh(gc + b_c)
    c_new = f_t * c + i_t * t_c
    o_t = jax.nn.sigmoid(go + w_c_o * c_new + b_o)
    cn[...] = c_new
    hn[...] = o_t * jnp.tanh(c_new)


@functools.lru_cache(maxsize=None)
def _make_dense_step(t0_flag):
    big = pl.BlockSpec((BLK, F_IN), lambda i: (i, 0))
    small = pl.BlockSpec((BLK, HID), lambda i: (i, 0))
    wxs = pl.BlockSpec((K * F_IN, 4 * HID), lambda i: (0, 0))
    whs = pl.BlockSpec((K * HID, 4 * HID), lambda i: (0, 0))
    pbs = pl.BlockSpec((8, HID), lambda i: (0, 0))
    return pl.pallas_call(
        functools.partial(_dense_step_body, t0_flag),
        grid=(GRID,),
        in_specs=[big] * 5 + [small] * 5 + [small, wxs, whs, pbs],
        out_specs=[small, small],
        out_shape=[jax.ShapeDtypeStruct((NP, HID), jnp.float32),
                   jax.ShapeDtypeStruct((NP, HID), jnp.float32)],
        name=f"dense_step_{t0_flag}",
    )


def _head_body(h_ref, wl, bl, out):
    out[...] = jnp.dot(jax.nn.relu(h_ref[...]), wl[...],
                       preferred_element_type=jnp.float32) + bl[...]


@functools.lru_cache(maxsize=None)
def _make_head():
    return pl.pallas_call(
        _head_body,
        grid=(GRID,),
        in_specs=[pl.BlockSpec((BLK, HID), lambda i: (i, 0)),
                  pl.BlockSpec((HID, 128), lambda i: (0, 0)),
                  pl.BlockSpec((1, 128), lambda i: (0, 0))],
        out_specs=pl.BlockSpec((BLK, 128), lambda i: (i, 0)),
        out_shape=jax.ShapeDtypeStruct((NP, 128), jnp.float32),
        name="head",
    )


def kernel(timesteps, edge_index, Wx_i, Wh_i, w_c_i, b_i, Wx_f, Wh_f, w_c_f, b_f,
           Wx_c, Wh_c, b_c, Wx_o, Wh_o, w_c_o, b_o, W_lin, b_lin):
    src = edge_index[0]
    dst = edge_index[1]
    mask = src != dst
    deg = jax.ops.segment_sum(mask.astype(jnp.float32), src, num_segments=N)
    dis = jnp.where(deg > 0, 1.0 / jnp.sqrt(jnp.maximum(deg, 1e-12)), 0.0)
    w = jnp.where(mask, -dis[src] * dis[dst], 0.0)

    order = jnp.argsort(dst)
    srcs = jnp.pad(src[order].astype(jnp.int32), (0, MAXE))
    dsts = jnp.pad(dst[order].astype(jnp.int32), (0, MAXE))
    ws = jnp.pad(w[order], (0, MAXE))
    bounds = jnp.arange(NW, dtype=jnp.int32) * ROWS
    starts = jnp.searchsorted(dsts, bounds, side="left").astype(jnp.int32)
    ends = jnp.searchsorted(dsts, bounds + ROWS, side="left").astype(jnp.int32)
    se = jnp.concatenate([starts, ends, jnp.zeros((32,), jnp.int32)])

    WxAll = jnp.concatenate([Wx_i, Wx_f, Wx_c, Wx_o], axis=2).reshape(K * F_IN, 4 * HID)
    WhAll = jnp.concatenate([Wh_i, Wh_f, Wh_c, Wh_o], axis=2).reshape(K * HID, 4 * HID)
    PB = jnp.concatenate([w_c_i, w_c_f, w_c_o, b_i, b_f, b_c, b_o,
                          jnp.zeros((1, HID), jnp.float32)], axis=0)

    xp = jnp.pad(timesteps, ((0, 0), (0, NP - N), (0, 0)))
    W_lin_p = jnp.pad(W_lin, ((0, 0), (0, 128 - W_lin.shape[1])))
    b_lin_p = jnp.pad(b_lin, ((0, 128 - b_lin.shape[0]),)).reshape(1, 128)

    zeros_h = jnp.zeros((NP, HID), jnp.float32)
    H = zeros_h
    C = zeros_h
    for t in range(T):
        txs, ths = _cheb2_stack(xp[t], H, srcs, dsts, ws, se)
        H, C = _make_dense_step(t == 0)(*txs, *ths, C, WxAll, WhAll, PB)

    out = _make_head()(H, W_lin_p, b_lin_p)
    return out[:N, :T]
